# baseline (speedup 1.0000x reference)
import functools

import jax
import jax.numpy as jnp
from jax import lax
from jax.experimental import pallas as pl
from jax.experimental.pallas import tpu as pltpu
from jax.experimental.pallas import tpu_sc as plsc

_R, _B, _D = 256, 32, 1024
_K = 6
_RBLK = 64
_NBLK = _R // _RBLK
_BB = _B // 2
_L = 16
_NCHUNK = _R // _L
_NC = 2


def _adj_body(h_ref, nm_ref, w_ref, th_ref, adj_ref):
    h = h_ref[...]
    nm = nm_ref[...]
    w = w_ref[...]
    s = jnp.dot(h.reshape(_RBLK * _BB, _D), w,
                preferred_element_type=jnp.float32).reshape(_RBLK, _BB)
    fb = jnp.sqrt(jnp.sum(nm * nm, axis=-1))
    th = th_ref[...]
    adj_ref[...] = s - th - 0.5 * ((1.0 - 0.9) * fb)


def _nms_sc_body(adj_hbm, hard_hbm, adj_v, sup_v, mask_v, cur_v):
    wid = lax.axis_index("s") * _NC + lax.axis_index("c")

    @pl.when(wid < _BB)
    def _():
        pltpu.sync_copy(adj_hbm.at[wid], adj_v)

        zero = jnp.zeros((_L,), jnp.float32)
        for c in range(_NCHUNK):
            sup_v[pl.ds(_L * c, _L)] = zero
            mask_v[pl.ds(_L * c, _L)] = zero

        iota = lax.iota(jnp.int32, _L)
        neg = jnp.full((_L,), -jnp.inf, jnp.float32)
        offs = jnp.where(iota == 1, 1, jnp.where(iota == 2, _R - 1, 0))
        ones = jnp.ones((_L,), jnp.float32)

        def bcast_max(x):
            return plsc.cummax(lax.rev(plsc.cummax(x), (0,)))

        for _t in range(_K):
            acc = neg
            for c in range(_NCHUNK):
                a = adj_v[pl.ds(_L * c, _L)]
                s = sup_v[pl.ds(_L * c, _L)]
                cur = jnp.where(s > 0, neg, a)
                cur_v[pl.ds(_L * c, _L)] = cur
                acc = jnp.maximum(acc, cur)
            m = bcast_max(acc)
            acci = jnp.full((_L,), 2 * _R, jnp.int32)
            for c in range(_NCHUNK):
                cur = cur_v[pl.ds(_L * c, _L)]
                cand = jnp.where(cur == m, iota + _L * c, 2 * _R)
                acci = jnp.minimum(acci, cand)
            idx = -bcast_max(-acci)
            idxvec = (idx + offs) % _R
            plsc.store_scatter(mask_v, [idxvec], ones, mask=iota < 1)
            plsc.store_scatter(sup_v, [idxvec], ones, mask=iota < 3)

        pltpu.sync_copy(mask_v, hard_hbm.at[wid])


def _scale_body(h_ref, m_ref, out_ref):
    out_ref[...] = h_ref[...] * m_ref[...][:, :, None]


def _make_adj_call(half):
    return pl.pallas_call(
        _adj_body,
        grid=(_NBLK,),
        in_specs=[
            pl.BlockSpec((_RBLK, _BB, _D), lambda i: (i, half, 0)),
            pl.BlockSpec((_RBLK, _BB, _D), lambda i: (i, half, 0)),
            pl.BlockSpec((_D, 1), lambda i: (0, 0)),
            pl.BlockSpec((_RBLK, 1), lambda i: (i, 0)),
        ],
        out_specs=pl.BlockSpec((_RBLK, _BB), lambda i: (i, 0)),
        out_shape=jax.ShapeDtypeStruct((_R, _BB), jnp.float32),
    )


def _make_scale_call(half, aliased):
    in_specs = [
        pl.BlockSpec((_RBLK, _BB, _D), lambda i: (i, half, 0)),
        pl.BlockSpec((_RBLK, _BB), lambda i: (i, 0)),
    ]
    if aliased:
        in_specs.append(pl.BlockSpec(memory_space=pltpu.MemorySpace.HBM))
    return pl.pallas_call(
        _scale_body if not aliased else _scale_body_aliased,
        grid=(_NBLK,),
        in_specs=in_specs,
        out_specs=pl.BlockSpec((_RBLK, _BB, _D), lambda i: (i, half, 0)),
        out_shape=jax.ShapeDtypeStruct((_R, _B, _D), jnp.float32),
        input_output_aliases={2: 0} if aliased else {},
    )


def _scale_body_aliased(h_ref, m_ref, prev_ref, out_ref):
    del prev_ref
    out_ref[...] = h_ref[...] * m_ref[...][:, :, None]


def kernel(H, neighbor_msg, W_score, theta):
    th2 = theta.reshape(_R, 1)

    adj_t1 = _make_adj_call(0)(H, neighbor_msg, W_score, th2)
    adj_t2 = _make_adj_call(1)(H, neighbor_msg, W_score, th2)

    nms = functools.partial(
        pl.kernel,
        mesh=plsc.VectorSubcoreMesh(core_axis_name="c", subcore_axis_name="s"),
        out_type=jax.ShapeDtypeStruct((_BB, _R), jnp.float32),
        scratch_types=[pltpu.VMEM((_R,), jnp.float32)] * 4,
        compiler_params=pltpu.CompilerParams(
            needs_layout_passes=False, use_tc_tiling_on_sc=False,
            skip_device_barrier=True),
    )(_nms_sc_body)

    hard1 = nms(adj_t1.T)
    hard2 = nms(adj_t2.T)

    hs1 = _make_scale_call(0, False)(H, hard1.T)
    Hs = _make_scale_call(1, True)(H, hard2.T, hs1)

    hard = jnp.concatenate([hard1, hard2], axis=0)
    adj = jnp.concatenate([adj_t1.T, adj_t2.T], axis=0)
    return (Hs, hard, adj)


# speedup vs baseline: 1.0752x; 1.0752x over previous
import functools

import jax
import jax.numpy as jnp
from jax import lax
from jax.experimental import pallas as pl
from jax.experimental.pallas import tpu as pltpu
from jax.experimental.pallas import tpu_sc as plsc

_R, _B, _D = 256, 32, 1024
_K = 6
_RBLK = 64
_NBLK = _R // _RBLK
_BB = _B // 2
_L = 16
_NCHUNK = _R // _L
_NC = 2


def _adj_body(h_ref, nm_ref, w_ref, th_ref, adj_ref):
    h = h_ref[...]
    nm = nm_ref[...]
    w = w_ref[...]
    s = jnp.dot(h.reshape(_RBLK * _BB, _D), w,
                preferred_element_type=jnp.float32).reshape(_RBLK, _BB)
    fb = jnp.sqrt(jnp.sum(nm * nm, axis=-1))
    th = th_ref[...]
    adj_ref[...] = s - th - 0.5 * ((1.0 - 0.9) * fb)


def _nms_sc_body(adj_hbm, hard_hbm, adj_v, sup_v, mask_v, cur_v):
    wid = lax.axis_index("s") * _NC + lax.axis_index("c")

    @pl.when(wid < _BB)
    def _():
        pltpu.sync_copy(adj_hbm.at[wid], adj_v)

        zero = jnp.zeros((_L,), jnp.float32)
        for c in range(_NCHUNK):
            sup_v[pl.ds(_L * c, _L)] = zero
            mask_v[pl.ds(_L * c, _L)] = zero

        iota = lax.iota(jnp.int32, _L)
        neg = jnp.full((_L,), -jnp.inf, jnp.float32)
        offs = jnp.where(iota == 1, 1, jnp.where(iota == 2, _R - 1, 0))
        ones = jnp.ones((_L,), jnp.float32)

        def bcast_max(x):
            return plsc.cummax(lax.rev(plsc.cummax(x), (0,)))

        for _t in range(_K):
            acc = neg
            for c in range(_NCHUNK):
                a = adj_v[pl.ds(_L * c, _L)]
                s = sup_v[pl.ds(_L * c, _L)]
                cur = jnp.where(s > 0, neg, a)
                cur_v[pl.ds(_L * c, _L)] = cur
                acc = jnp.maximum(acc, cur)
            m = bcast_max(acc)
            acci = jnp.full((_L,), 2 * _R, jnp.int32)
            for c in range(_NCHUNK):
                cur = cur_v[pl.ds(_L * c, _L)]
                cand = jnp.where(cur == m, iota + _L * c, 2 * _R)
                acci = jnp.minimum(acci, cand)
            idx = -bcast_max(-acci)
            idxvec = (idx + offs) % _R
            plsc.store_scatter(mask_v, [idxvec], ones, mask=iota < 1)
            plsc.store_scatter(sup_v, [idxvec], ones, mask=iota < 3)

        pltpu.sync_copy(mask_v, hard_hbm.at[wid])


def _scale_body(h_ref, m_ref, out_ref):
    out_ref[...] = h_ref[...] * m_ref[...][:, :, None]


def _make_adj_call(half):
    return pl.pallas_call(
        _adj_body,
        grid=(_NBLK,),
        in_specs=[
            pl.BlockSpec((_RBLK, _BB, _D), lambda i: (i, half, 0)),
            pl.BlockSpec((_RBLK, _BB, _D), lambda i: (i, half, 0)),
            pl.BlockSpec((_D, 1), lambda i: (0, 0)),
            pl.BlockSpec((_RBLK, 1), lambda i: (i, 0)),
        ],
        out_specs=pl.BlockSpec((_RBLK, _BB), lambda i: (i, 0)),
        out_shape=jax.ShapeDtypeStruct((_R, _BB), jnp.float32),
    )


def _make_scale_call():
    return pl.pallas_call(
        _scale_body,
        grid=(_NBLK,),
        in_specs=[
            pl.BlockSpec((_RBLK, _B, _D), lambda i: (i, 0, 0)),
            pl.BlockSpec((_RBLK, _B), lambda i: (i, 0)),
        ],
        out_specs=pl.BlockSpec((_RBLK, _B, _D), lambda i: (i, 0, 0)),
        out_shape=jax.ShapeDtypeStruct((_R, _B, _D), jnp.float32),
    )


def kernel(H, neighbor_msg, W_score, theta):
    th2 = theta.reshape(_R, 1)

    adj_t1 = _make_adj_call(0)(H, neighbor_msg, W_score, th2)
    adj_t2 = _make_adj_call(1)(H, neighbor_msg, W_score, th2)

    nms = functools.partial(
        pl.kernel,
        mesh=plsc.VectorSubcoreMesh(core_axis_name="c", subcore_axis_name="s"),
        out_type=jax.ShapeDtypeStruct((_BB, _R), jnp.float32),
        scratch_types=[pltpu.VMEM((_R,), jnp.float32)] * 4,
        compiler_params=pltpu.CompilerParams(
            needs_layout_passes=False, use_tc_tiling_on_sc=False,
            skip_device_barrier=True),
    )(_nms_sc_body)

    hard1 = nms(adj_t1.T)
    hard2 = nms(adj_t2.T)

    hard = jnp.concatenate([hard1, hard2], axis=0)
    Hs = _make_scale_call()(H, hard.T)

    adj = jnp.concatenate([adj_t1.T, adj_t2.T], axis=0)
    return (Hs, hard, adj)


# speedup vs baseline: 1.1185x; 1.0402x over previous
import functools

import jax
import jax.numpy as jnp
from jax import lax
from jax.experimental import pallas as pl
from jax.experimental.pallas import tpu as pltpu
from jax.experimental.pallas import tpu_sc as plsc

_R, _B, _D = 256, 32, 1024
_K = 6
_RBLK = 64
_NBLK = _R // _RBLK
_L = 16
_NCHUNK = _R // _L
_NC = 2


def _adj_body(h_ref, nm_ref, w_ref, th_ref, adj_ref):
    h = h_ref[...]
    nm = nm_ref[...]
    w = w_ref[...]
    s = jnp.dot(h.reshape(_RBLK * _B, _D), w,
                preferred_element_type=jnp.float32).reshape(_RBLK, _B)
    fb = jnp.sqrt(jnp.sum(nm * nm, axis=-1))
    th = th_ref[...]
    adj_ref[...] = s - th - 0.5 * ((1.0 - 0.9) * fb)


def _nms_sc_body(adj_hbm, hard_hbm, adj_v, sup_v, mask_v, cur_v):
    wid = lax.axis_index("s") * _NC + lax.axis_index("c")
    pltpu.sync_copy(adj_hbm.at[wid], adj_v)

    zero = jnp.zeros((_L,), jnp.float32)
    for c in range(_NCHUNK):
        sup_v[pl.ds(_L * c, _L)] = zero
        mask_v[pl.ds(_L * c, _L)] = zero

    iota = lax.iota(jnp.int32, _L)
    neg = jnp.full((_L,), -jnp.inf, jnp.float32)
    offs = jnp.where(iota == 1, 1, jnp.where(iota == 2, _R - 1, 0))
    ones = jnp.ones((_L,), jnp.float32)

    def bcast_max(x):
        return plsc.cummax(lax.rev(plsc.cummax(x), (0,)))

    for _ in range(_K):
        acc = neg
        for c in range(_NCHUNK):
            a = adj_v[pl.ds(_L * c, _L)]
            s = sup_v[pl.ds(_L * c, _L)]
            cur = jnp.where(s > 0, neg, a)
            cur_v[pl.ds(_L * c, _L)] = cur
            acc = jnp.maximum(acc, cur)
        m = bcast_max(acc)
        acci = jnp.full((_L,), 2 * _R, jnp.int32)
        for c in range(_NCHUNK):
            cur = cur_v[pl.ds(_L * c, _L)]
            cand = jnp.where(cur == m, iota + _L * c, 2 * _R)
            acci = jnp.minimum(acci, cand)
        idx = -bcast_max(-acci)
        idxvec = (idx + offs) % _R
        plsc.store_scatter(mask_v, [idxvec], ones, mask=iota < 1)
        plsc.store_scatter(sup_v, [idxvec], ones, mask=iota < 3)

    pltpu.sync_copy(mask_v, hard_hbm.at[wid])


def _scale_body(h_ref, m_ref, out_ref):
    out_ref[...] = h_ref[...] * m_ref[...][:, :, None]


def kernel(H, neighbor_msg, W_score, theta):
    adj_t = pl.pallas_call(
        _adj_body,
        grid=(_NBLK,),
        in_specs=[
            pl.BlockSpec((_RBLK, _B, _D), lambda i: (i, 0, 0)),
            pl.BlockSpec((_RBLK, _B, _D), lambda i: (i, 0, 0)),
            pl.BlockSpec((_D, 1), lambda i: (0, 0)),
            pl.BlockSpec((_RBLK, 1), lambda i: (i, 0)),
        ],
        out_specs=pl.BlockSpec((_RBLK, _B), lambda i: (i, 0)),
        out_shape=jax.ShapeDtypeStruct((_R, _B), jnp.float32),
    )(H, neighbor_msg, W_score, theta.reshape(_R, 1))

    adj = adj_t.T

    nms = functools.partial(
        pl.kernel,
        mesh=plsc.VectorSubcoreMesh(core_axis_name="c", subcore_axis_name="s"),
        out_type=jax.ShapeDtypeStruct((_B, _R), jnp.float32),
        scratch_types=[pltpu.VMEM((_R,), jnp.float32)] * 4,
        compiler_params=pltpu.CompilerParams(
            needs_layout_passes=False, use_tc_tiling_on_sc=False,
            skip_device_barrier=True),
    )(_nms_sc_body)
    hard = nms(adj)

    Hs = pl.pallas_call(
        _scale_body,
        grid=(_NBLK,),
        in_specs=[
            pl.BlockSpec((_RBLK, _B, _D), lambda i: (i, 0, 0)),
            pl.BlockSpec((_RBLK, _B), lambda i: (i, 0)),
        ],
        out_specs=pl.BlockSpec((_RBLK, _B, _D), lambda i: (i, 0, 0)),
        out_shape=jax.ShapeDtypeStruct((_R, _B, _D), jnp.float32),
    )(H, hard.T)

    return (Hs, hard, adj)


# speedup vs baseline: 1.1187x; 1.0002x over previous
import functools

import jax
import jax.numpy as jnp
from jax import lax
from jax.experimental import pallas as pl
from jax.experimental.pallas import tpu as pltpu
from jax.experimental.pallas import tpu_sc as plsc

_R, _B, _D = 256, 32, 1024
_K = 6
_RBLK = 64
_NBLK = _R // _RBLK
_L = 16
_NCHUNK = _R // _L
_NC = 2


def _adj_body(h_ref, nm_ref, w_ref, th_ref, adj_ref):
    h = h_ref[...]
    nm = nm_ref[...]
    w = w_ref[...]
    s = jnp.dot(h.reshape(_RBLK * _B, _D), w,
                preferred_element_type=jnp.float32).reshape(_RBLK, _B)
    fb = jnp.sqrt(jnp.sum(nm * nm, axis=-1))
    th = th_ref[...]
    adj_ref[...] = s - th - 0.5 * ((1.0 - 0.9) * fb)


def _nms_sc_body(adj_hbm, hard_hbm, adj_v, sup_v, mask_v, cur_v):
    wid = lax.axis_index("s") * _NC + lax.axis_index("c")
    pltpu.sync_copy(adj_hbm.at[wid], adj_v)

    zero = jnp.zeros((_L,), jnp.float32)
    for c in range(_NCHUNK):
        sup_v[pl.ds(_L * c, _L)] = zero
        mask_v[pl.ds(_L * c, _L)] = zero

    iota = lax.iota(jnp.int32, _L)
    neg = jnp.full((_L,), -jnp.inf, jnp.float32)
    offs = jnp.where(iota == 1, 1, jnp.where(iota == 2, _R - 1, 0))
    ones = jnp.ones((_L,), jnp.float32)

    def bcast_max(x):
        return plsc.cummax(lax.rev(plsc.cummax(x), (0,)))

    for _ in range(_K):
        acc = neg
        for c in range(_NCHUNK):
            a = adj_v[pl.ds(_L * c, _L)]
            s = sup_v[pl.ds(_L * c, _L)]
            cur = jnp.where(s > 0, neg, a)
            cur_v[pl.ds(_L * c, _L)] = cur
            acc = jnp.maximum(acc, cur)
        m = bcast_max(acc)
        acci = jnp.full((_L,), 2 * _R, jnp.int32)
        for c in range(_NCHUNK):
            cur = cur_v[pl.ds(_L * c, _L)]
            cand = jnp.where(cur == m, iota + _L * c, 2 * _R)
            acci = jnp.minimum(acci, cand)
        idx = -bcast_max(-acci)
        idxvec = (idx + offs) % _R
        plsc.store_scatter(mask_v, [idxvec], ones, mask=iota < 1)
        plsc.store_scatter(sup_v, [idxvec], ones, mask=iota < 3)

    pltpu.sync_copy(mask_v, hard_hbm.at[wid])


def _scale_body(h_ref, m_ref, out_ref):
    out_ref[...] = h_ref[...] * m_ref[...][:, :, None]


def kernel(H, neighbor_msg, W_score, theta):
    adj_t = pl.pallas_call(
        _adj_body,
        grid=(_NBLK,),
        in_specs=[
            pl.BlockSpec((_RBLK, _B, _D), lambda i: (i, 0, 0)),
            pl.BlockSpec((_RBLK, _B, _D), lambda i: (i, 0, 0)),
            pl.BlockSpec((_D, 1), lambda i: (0, 0)),
            pl.BlockSpec((_RBLK, 1), lambda i: (i, 0)),
        ],
        out_specs=pl.BlockSpec((_RBLK, _B), lambda i: (i, 0)),
        out_shape=jax.ShapeDtypeStruct((_R, _B), jnp.float32),
    )(H, neighbor_msg, W_score, theta.reshape(_R, 1))

    adj = adj_t.T

    nms = functools.partial(
        pl.kernel,
        mesh=plsc.VectorSubcoreMesh(core_axis_name="c", subcore_axis_name="s"),
        out_type=jax.ShapeDtypeStruct((_B, _R), jnp.float32),
        scratch_types=[pltpu.VMEM((_R,), jnp.float32)] * 4,
        compiler_params=pltpu.CompilerParams(
            needs_layout_passes=False, use_tc_tiling_on_sc=False),
    )(_nms_sc_body)
    hard = nms(adj)

    Hs = pl.pallas_call(
        _scale_body,
        grid=(_NBLK,),
        in_specs=[
            pl.BlockSpec((_RBLK, _B, _D), lambda i: (i, 0, 0)),
            pl.BlockSpec((_RBLK, _B), lambda i: (i, 0)),
        ],
        out_specs=pl.BlockSpec((_RBLK, _B, _D), lambda i: (i, 0, 0)),
        out_shape=jax.ShapeDtypeStruct((_R, _B, _D), jnp.float32),
    )(H, hard.T)

    return (Hs, hard, adj)
